# Initial kernel scaffold; baseline (speedup 1.0000x reference)
import jax
import jax.numpy as jnp
from jax.experimental import pallas as pl


def kernel(x, edge_index, batch, W1, a1s, a1d, b1, W2, a2s, a2d, b2, W3, a3s, a3d, b3, bn_g, bn_b, bn_m, bn_v, fc1_w, fc1_b, fc2_w, fc2_b):
    raise NotImplementedError("write your pallas kernel here")


# speedup vs baseline: 55.9000x; 55.9000x over previous
import functools
import jax
import jax.numpy as jnp
from jax import lax
from jax.experimental import pallas as pl
from jax.experimental.pallas import tpu as pltpu
from jax.experimental.pallas import tpu_sc as plsc

N = 10000
E = 320000
H = 8
G = 128
NEG = 0.2
CH = 128
NCHUNK = E // CH
NW = 32
ITERS = -(-NCHUNK // NW)
RPT = N // 16


def _front_body(x_ref, w_ref, a_ref, h_ref, asd_ref):
    h = jnp.dot(x_ref[...], w_ref[...], preferred_element_type=jnp.float32)
    h_ref[...] = h
    asd_ref[...] = jnp.dot(h, a_ref[...], preferred_element_type=jnp.float32)


def _tc_front(x, w, asd_w, bm=1000):
    fin = x.shape[1]
    hc = w.shape[1]
    grid = x.shape[0] // bm
    return pl.pallas_call(
        _front_body,
        grid=(grid,),
        in_specs=[
            pl.BlockSpec((bm, fin), lambda i: (i, 0)),
            pl.BlockSpec((fin, hc), lambda i: (0, 0)),
            pl.BlockSpec((hc, 16), lambda i: (0, 0)),
        ],
        out_specs=[
            pl.BlockSpec((bm, hc), lambda i: (i, 0)),
            pl.BlockSpec((bm, 16), lambda i: (i, 0)),
        ],
        out_shape=[
            jax.ShapeDtypeStruct((x.shape[0], hc), jnp.float32),
            jax.ShapeDtypeStruct((x.shape[0], 16), jnp.float32),
        ],
    )(x, w, asd_w)


def _combine(p0, p1, h, asd, rm, b, hc):
    num = p0[:, :hc] + p1[:, :hc]
    den8 = p0[:, hc:hc + 8] + p1[:, hc:hc + 8]
    als = asd[:, :8]
    ald = asd[:, 8:]
    es = als + ald
    es = jnp.maximum(es, NEG * es)
    exs = jnp.exp(es)
    exs_e = jnp.dot(exs, rm, preferred_element_type=jnp.float32)
    num = num + exs_e * h
    den = jnp.dot(den8 + exs, rm, preferred_element_type=jnp.float32)
    out = num / (den + 1e-16) + b
    return jnp.where(out > 0, out, jnp.exp(jnp.minimum(out, 0.0)) - 1.0)


def _mid_body(p0_ref, p1_ref, h_ref, asd_ref, rm_ref, b_ref, w_ref, a_ref,
              hn_ref, asdn_ref, *, hc):
    act = _combine(p0_ref[...], p1_ref[...], h_ref[...], asd_ref[...],
                   rm_ref[...], b_ref[...], hc)
    hn = jnp.dot(act, w_ref[...], preferred_element_type=jnp.float32)
    hn_ref[...] = hn
    asdn_ref[...] = jnp.dot(hn, a_ref[...], preferred_element_type=jnp.float32)


def _tc_mid(p0, p1, h, asd, rm, b, w, asd_w, bm=1000):
    hc = h.shape[1]
    wl = p0.shape[1]
    hc2 = w.shape[1]
    grid = N // bm
    return pl.pallas_call(
        functools.partial(_mid_body, hc=hc),
        grid=(grid,),
        in_specs=[
            pl.BlockSpec((bm, wl), lambda i: (i, 0)),
            pl.BlockSpec((bm, wl), lambda i: (i, 0)),
            pl.BlockSpec((bm, hc), lambda i: (i, 0)),
            pl.BlockSpec((bm, 16), lambda i: (i, 0)),
            pl.BlockSpec((8, hc), lambda i: (0, 0)),
            pl.BlockSpec((1, hc), lambda i: (0, 0)),
            pl.BlockSpec((hc, hc2), lambda i: (0, 0)),
            pl.BlockSpec((hc2, 16), lambda i: (0, 0)),
        ],
        out_specs=[
            pl.BlockSpec((bm, hc2), lambda i: (i, 0)),
            pl.BlockSpec((bm, 16), lambda i: (i, 0)),
        ],
        out_shape=[
            jax.ShapeDtypeStruct((N, hc2), jnp.float32),
            jax.ShapeDtypeStruct((N, 16), jnp.float32),
        ],
    )(p0, p1, h, asd, rm, b, w, asd_w)


def _pool_body(p0_ref, p1_ref, h_ref, asd_ref, rm_ref, b_ref, bt_ref,
               sum_ref, cnt_ref, *, hc, bm):
    act = _combine(p0_ref[...], p1_ref[...], h_ref[...], asd_ref[...],
                   rm_ref[...], b_ref[...], hc)
    gi = lax.broadcasted_iota(jnp.int32, (bm, G), 1)
    oh = (bt_ref[...] == gi).astype(jnp.float32)
    dn = (((0,), (0,)), ((), ()))
    s = lax.dot_general(oh, act, dn, preferred_element_type=jnp.float32)
    c = lax.dot_general(oh, jnp.ones_like(act), dn,
                        preferred_element_type=jnp.float32)
    first = pl.program_id(0) == 0

    @pl.when(first)
    def _():
        sum_ref[...] = s
        cnt_ref[...] = c

    @pl.when(jnp.logical_not(first))
    def _():
        sum_ref[...] += s
        cnt_ref[...] += c


def _tc_pool(p0, p1, h, asd, rm, b, batch2d, bm=1000):
    hc = h.shape[1]
    wl = p0.shape[1]
    grid = N // bm
    return pl.pallas_call(
        functools.partial(_pool_body, hc=hc, bm=bm),
        grid=(grid,),
        in_specs=[
            pl.BlockSpec((bm, wl), lambda i: (i, 0)),
            pl.BlockSpec((bm, wl), lambda i: (i, 0)),
            pl.BlockSpec((bm, hc), lambda i: (i, 0)),
            pl.BlockSpec((bm, 16), lambda i: (i, 0)),
            pl.BlockSpec((8, hc), lambda i: (0, 0)),
            pl.BlockSpec((1, hc), lambda i: (0, 0)),
            pl.BlockSpec((bm, 1), lambda i: (i, 0)),
        ],
        out_specs=[
            pl.BlockSpec((G, hc), lambda i: (0, 0)),
            pl.BlockSpec((G, hc), lambda i: (0, 0)),
        ],
        out_shape=[
            jax.ShapeDtypeStruct((G, hc), jnp.float32),
            jax.ShapeDtypeStruct((G, hc), jnp.float32),
        ],
    )(p0, p1, h, asd, rm, b, batch2d)


def _head_body(sum_ref, cnt_ref, bnm_ref, bnv_ref, bng_ref, bnb_ref,
               f1w_ref, f1b_ref, f2w_ref, f2b_ref, out_ref):
    pooled = sum_ref[...] / jnp.maximum(cnt_ref[...], 1.0)
    y = (pooled - bnm_ref[...]) / jnp.sqrt(bnv_ref[...] + 1e-5)
    y = y * bng_ref[...] + bnb_ref[...]
    z = jnp.dot(y, f1w_ref[...], preferred_element_type=jnp.float32)
    z = jnp.maximum(z + f1b_ref[...], 0.0)
    lg = jnp.dot(z, f2w_ref[...], preferred_element_type=jnp.float32)
    lg = lg + f2b_ref[...]
    m = jnp.max(lg, axis=1, keepdims=True)
    lse = jnp.log(jnp.sum(jnp.exp(lg - m), axis=1, keepdims=True))
    out_ref[...] = lg - m - lse


def _tc_head(sums, cnt, bn_m, bn_v, bn_g, bn_b, f1w, f1b, f2w, f2b):
    return pl.pallas_call(
        _head_body,
        out_shape=jax.ShapeDtypeStruct((G, 10), jnp.float32),
    )(sums, cnt, bn_m, bn_v, bn_g, bn_b, f1w, f1b, f2w, f2b)


def _lane_bcast(v, idx):
    dn = lax.GatherDimensionNumbers(
        offset_dims=(), collapsed_slice_dims=(0,), start_index_map=(0,))
    return lax.gather(v, idx[:, None], dn, (1,),
                      mode=lax.GatherScatterMode.PROMISE_IN_BOUNDS)


def _sc_body(h_hbm, asd_hbm, src_hbm, dst_hbm, out_hbm,
             sidx, didx, asr, adr, hr, wtd, sem_s, sem_d, sem_h, acc, *, hc):
    wl = hc + 16
    c = lax.axis_index("c")
    s = lax.axis_index("s")
    wid = s * 2 + c
    lanes = lax.iota(jnp.int32, 16)
    rot = (lanes & 7) + 8

    def zb(r, _):
        for k in range(wl // 16):
            wtd[r, k * 16:(k + 1) * 16] = jnp.zeros((16,), jnp.float32)
        return 0

    lax.fori_loop(0, CH, zb, 0)
    base = s * RPT
    for j in range(5):
        pltpu.sync_copy(wtd.at[pl.ds(0, RPT // 5)],
                        acc.at[pl.ds(base + j * (RPT // 5), RPT // 5)])
    plsc.subcore_barrier()

    def chunk(it, _):
        cid = it * NW + wid

        @pl.when(cid < NCHUNK)
        def _():
            off = cid * CH
            pltpu.sync_copy(src_hbm.at[pl.ds(off, CH)], sidx)
            pltpu.sync_copy(dst_hbm.at[pl.ds(off, CH)], didx)
            cp1 = pltpu.async_copy(asd_hbm.at[sidx], asr, sem_s)
            cp2 = pltpu.async_copy(asd_hbm.at[didx], adr, sem_d)
            cp3 = pltpu.async_copy(h_hbm.at[sidx], hr, sem_h)
            cp1.wait()
            cp2.wait()
            cp3.wait()

            def edge(e, _):
                av = asr[e]
                dv = _lane_bcast(adr[e], rot)
                ev = av + dv
                ev = jnp.maximum(ev, NEG * ev)
                exv = jnp.exp(ev)
                wtd[e, pl.ds(hc, 16)] = exv
                for j in range(hc // 16):
                    mj = _lane_bcast(exv, _head_idx(j, hc))
                    wtd[e, pl.ds(j * 16, 16)] = hr[e, pl.ds(j * 16, 16)] * mj
                return 0

            lax.fori_loop(0, CH, edge, 0)
            pltpu.sync_copy(wtd, acc.at[didx], add=True)
        return 0

    lax.fori_loop(0, ITERS, chunk, 0)
    plsc.subcore_barrier()
    for j in range(5):
        sl = pl.ds(base + j * (RPT // 5), RPT // 5)
        pltpu.sync_copy(acc.at[sl], out_hbm.at[c].at[sl])


def _head_idx(j, hc):
    lanes = lax.iota(jnp.int32, 16)
    if hc == 128:
        return jnp.full((16,), j, jnp.int32)
    return 2 * j + (lanes >> 3)


def _sc_edge(h, asd, src, dst):
    hc = h.shape[1]
    wl = hc + 16
    mesh = plsc.VectorSubcoreMesh(core_axis_name="c", subcore_axis_name="s")
    kern = pl.kernel(
        functools.partial(_sc_body, hc=hc),
        out_type=jax.ShapeDtypeStruct((2, N, wl), jnp.float32),
        mesh=mesh,
        compiler_params=pltpu.CompilerParams(use_tc_tiling_on_sc=False),
        scratch_types=[
            pltpu.VMEM((CH,), jnp.int32),
            pltpu.VMEM((CH,), jnp.int32),
            pltpu.VMEM((CH, 16), jnp.float32),
            pltpu.VMEM((CH, 16), jnp.float32),
            pltpu.VMEM((CH, hc), jnp.float32),
            pltpu.VMEM((CH, wl), jnp.float32),
            pltpu.SemaphoreType.DMA,
            pltpu.SemaphoreType.DMA,
            pltpu.SemaphoreType.DMA,
            pltpu.VMEM_SHARED((N, wl), jnp.float32),
        ],
    )
    return kern(h, asd, src, dst)


def _block_diag_asd(a_s, a_d):
    hh, cc = a_s.shape
    m = jnp.zeros((hh * cc, 2 * hh), jnp.float32)
    rows = jnp.arange(hh * cc)
    heads = rows // cc
    m = m.at[rows, heads].set(a_s.reshape(-1))
    m = m.at[rows, hh + heads].set(a_d.reshape(-1))
    return m


def _rep_mat(hc):
    return jnp.repeat(jnp.eye(H, dtype=jnp.float32), hc // H, axis=1)


def kernel(x, edge_index, batch, W1, a1s, a1d, b1, W2, a2s, a2d, b2,
           W3, a3s, a3d, b3, bn_g, bn_b, bn_m, bn_v,
           fc1_w, fc1_b, fc2_w, fc2_b):
    src = edge_index[0]
    dst = edge_index[1]
    asd1_w = _block_diag_asd(a1s, a1d)
    asd2_w = _block_diag_asd(a2s, a2d)
    asd3_w = _block_diag_asd(a3s, a3d)
    rm1 = _rep_mat(64)
    rm2 = _rep_mat(128)
    batch2d = batch.reshape(N, 1)

    h1, asd1 = _tc_front(x, W1, asd1_w)
    p1 = _sc_edge(h1, asd1, src, dst)
    h2, asd2 = _tc_mid(p1[0], p1[1], h1, asd1, rm1, b1.reshape(1, 64),
                       W2, asd2_w)
    p2 = _sc_edge(h2, asd2, src, dst)
    h3, asd3 = _tc_mid(p2[0], p2[1], h2, asd2, rm2, b2.reshape(1, 128),
                       W3, asd3_w)
    p3 = _sc_edge(h3, asd3, src, dst)
    sums, cnt = _tc_pool(p3[0], p3[1], h3, asd3, rm2, b3.reshape(1, 128),
                         batch2d)
    return _tc_head(sums, cnt, bn_m.reshape(1, 128), bn_v.reshape(1, 128),
                    bn_g.reshape(1, 128), bn_b.reshape(1, 128),
                    fc1_w, fc1_b.reshape(1, 32), fc2_w, fc2_b.reshape(1, 10))


# speedup vs baseline: 170.8013x; 3.0555x over previous
import functools
import jax
import jax.numpy as jnp
from jax import lax
from jax.experimental import pallas as pl
from jax.experimental.pallas import tpu as pltpu
from jax.experimental.pallas import tpu_sc as plsc

N = 10000
E = 320000
H = 8
G = 128
NEG = 0.2
NW = 32
CH = 50
NCH = E // (NW * CH)
GC = 10
NG = NCH // GC
RPT = N // 16


def _front_body(x_ref, w_ref, a_ref, ht_ref, asd_ref, *, hc):
    h = jnp.dot(x_ref[...], w_ref[...], preferred_element_type=jnp.float32)
    asd = jnp.dot(h, a_ref[...], preferred_element_type=jnp.float32)
    ht_ref[:, :hc] = h
    ht_ref[:, hc:] = asd
    asd_ref[...] = asd


def _tc_front(x, w, asd_w, bm=2000):
    fin = x.shape[1]
    hc = w.shape[1]
    grid = x.shape[0] // bm
    return pl.pallas_call(
        functools.partial(_front_body, hc=hc),
        grid=(grid,),
        in_specs=[
            pl.BlockSpec((bm, fin), lambda i: (i, 0)),
            pl.BlockSpec((fin, hc), lambda i: (0, 0)),
            pl.BlockSpec((hc, 16), lambda i: (0, 0)),
        ],
        out_specs=[
            pl.BlockSpec((bm, hc + 16), lambda i: (i, 0)),
            pl.BlockSpec((bm, 16), lambda i: (i, 0)),
        ],
        out_shape=[
            jax.ShapeDtypeStruct((x.shape[0], hc + 16), jnp.float32),
            jax.ShapeDtypeStruct((x.shape[0], 16), jnp.float32),
        ],
    )(x, w, asd_w)


def _combine(p0, p1, ht, rm, b, hc):
    h = ht[:, :hc]
    als = ht[:, hc:hc + 8]
    ald = ht[:, hc + 8:hc + 16]
    num = p0[:, :hc] + p1[:, :hc]
    den8 = p0[:, hc:hc + 8] + p1[:, hc:hc + 8]
    es = als + ald
    es = jnp.maximum(es, NEG * es)
    exs = jnp.exp(es)
    exs_e = jnp.dot(exs, rm, preferred_element_type=jnp.float32)
    num = num + exs_e * h
    den = jnp.dot(den8 + exs, rm, preferred_element_type=jnp.float32)
    out = num / (den + 1e-16) + b
    return jnp.where(out > 0, out, jnp.exp(jnp.minimum(out, 0.0)) - 1.0)


def _mid_body(p0_ref, p1_ref, ht_ref, rm_ref, b_ref, w_ref, a_ref,
              htn_ref, asdn_ref, *, hc, hc2):
    act = _combine(p0_ref[...], p1_ref[...], ht_ref[...],
                   rm_ref[...], b_ref[...], hc)
    hn = jnp.dot(act, w_ref[...], preferred_element_type=jnp.float32)
    asdn = jnp.dot(hn, a_ref[...], preferred_element_type=jnp.float32)
    htn_ref[:, :hc2] = hn
    htn_ref[:, hc2:] = asdn
    asdn_ref[...] = asdn


def _tc_mid(p, ht, rm, b, w, asd_w, bm=2000):
    hc = ht.shape[1] - 16
    wl = hc + 16
    hc2 = w.shape[1]
    grid = N // bm
    return pl.pallas_call(
        functools.partial(_mid_body, hc=hc, hc2=hc2),
        grid=(grid,),
        in_specs=[
            pl.BlockSpec((bm, wl), lambda i: (i, 0)),
            pl.BlockSpec((bm, wl), lambda i: (i, 0)),
            pl.BlockSpec((bm, hc + 16), lambda i: (i, 0)),
            pl.BlockSpec((8, hc), lambda i: (0, 0)),
            pl.BlockSpec((1, hc), lambda i: (0, 0)),
            pl.BlockSpec((hc, hc2), lambda i: (0, 0)),
            pl.BlockSpec((hc2, 16), lambda i: (0, 0)),
        ],
        out_specs=[
            pl.BlockSpec((bm, hc2 + 16), lambda i: (i, 0)),
            pl.BlockSpec((bm, 16), lambda i: (i, 0)),
        ],
        out_shape=[
            jax.ShapeDtypeStruct((N, hc2 + 16), jnp.float32),
            jax.ShapeDtypeStruct((N, 16), jnp.float32),
        ],
    )(p[0], p[1], ht, rm, b, w, asd_w)


def _pool_body(p0_ref, p1_ref, ht_ref, rm_ref, b_ref, bt_ref,
               bnm_ref, bnv_ref, bng_ref, bnb_ref,
               f1w_ref, f1b_ref, f2w_ref, f2b_ref,
               out_ref, sum_ref, cnt_ref, *, hc, bm):
    act = _combine(p0_ref[...], p1_ref[...], ht_ref[...],
                   rm_ref[...], b_ref[...], hc)
    gi = lax.broadcasted_iota(jnp.int32, (bm, G), 1)
    oh = (bt_ref[...] == gi).astype(jnp.float32)
    dn = (((0,), (0,)), ((), ()))
    s = lax.dot_general(oh, act, dn, preferred_element_type=jnp.float32)
    c = lax.dot_general(oh, jnp.ones_like(act), dn,
                        preferred_element_type=jnp.float32)
    step = pl.program_id(0)

    @pl.when(step == 0)
    def _():
        sum_ref[...] = s
        cnt_ref[...] = c

    @pl.when(step != 0)
    def _():
        sum_ref[...] += s
        cnt_ref[...] += c

    @pl.when(step == pl.num_programs(0) - 1)
    def _():
        pooled = sum_ref[...] / jnp.maximum(cnt_ref[...], 1.0)
        y = (pooled - bnm_ref[...]) / jnp.sqrt(bnv_ref[...] + 1e-5)
        y = y * bng_ref[...] + bnb_ref[...]
        z = jnp.dot(y, f1w_ref[...], preferred_element_type=jnp.float32)
        z = jnp.maximum(z + f1b_ref[...], 0.0)
        lg = jnp.dot(z, f2w_ref[...], preferred_element_type=jnp.float32)
        lg = lg + f2b_ref[...]
        m = jnp.max(lg, axis=1, keepdims=True)
        lse = jnp.log(jnp.sum(jnp.exp(lg - m), axis=1, keepdims=True))
        out_ref[...] = lg - m - lse


def _tc_pool_head(p, ht, rm, b, batch2d, bn_m, bn_v, bn_g, bn_b,
                  f1w, f1b, f2w, f2b, bm=2000):
    hc = ht.shape[1] - 16
    wl = hc + 16
    grid = N // bm
    cst = lambda i: (0, 0)
    return pl.pallas_call(
        functools.partial(_pool_body, hc=hc, bm=bm),
        grid=(grid,),
        in_specs=[
            pl.BlockSpec((bm, wl), lambda i: (i, 0)),
            pl.BlockSpec((bm, wl), lambda i: (i, 0)),
            pl.BlockSpec((bm, hc + 16), lambda i: (i, 0)),
            pl.BlockSpec((8, hc), cst),
            pl.BlockSpec((1, hc), cst),
            pl.BlockSpec((bm, 1), lambda i: (i, 0)),
            pl.BlockSpec((1, hc), cst),
            pl.BlockSpec((1, hc), cst),
            pl.BlockSpec((1, hc), cst),
            pl.BlockSpec((1, hc), cst),
            pl.BlockSpec((hc, 32), cst),
            pl.BlockSpec((1, 32), cst),
            pl.BlockSpec((32, 10), cst),
            pl.BlockSpec((1, 10), cst),
        ],
        out_specs=pl.BlockSpec((G, 10), cst),
        out_shape=jax.ShapeDtypeStruct((G, 10), jnp.float32),
        scratch_shapes=[
            pltpu.VMEM((G, hc), jnp.float32),
            pltpu.VMEM((G, hc), jnp.float32),
        ],
    )(p[0], p[1], ht, rm, b, batch2d, bn_m, bn_v, bn_g, bn_b,
      f1w, f1b, f2w, f2b)


def _lane_bcast(v, idx):
    dn = lax.GatherDimensionNumbers(
        offset_dims=(), collapsed_slice_dims=(0,), start_index_map=(0,))
    return lax.gather(v, idx[:, None], dn, (1,),
                      mode=lax.GatherScatterMode.PROMISE_IN_BOUNDS)


def _head_idx(j, hc):
    lanes = lax.iota(jnp.int32, 16)
    if hc == 128:
        return jnp.full((16,), j, jnp.int32)
    return 2 * j + (lanes >> 3)


def _sc_body(ht_hbm, asd_hbm, src_hbm, dst_hbm, out_hbm,
             sidx, didx, hta0, adr0, wtd0, hta1, adr1, wtd1,
             sg0, sg1, sw0, sw1, si, acc, *, hc):
    wl = hc + 16
    c = lax.axis_index("c")
    s = lax.axis_index("s")
    wid = s * 2 + c
    lanes = lax.iota(jnp.int32, 16)
    rot = (lanes & 7) + 8
    hta = (hta0, hta1)
    adr = (adr0, adr1)
    wtd = (wtd0, wtd1)
    sg = (sg0, sg1)
    sw = (sw0, sw1)

    def irow(j):
        return ((j // GC) % 2) * GC + (j % GC)

    def zb(r, _):
        for k in range(wl // 16):
            wtd0[r, k * 16:(k + 1) * 16] = jnp.zeros((16,), jnp.float32)
        return 0

    lax.fori_loop(0, CH, zb, 0)
    base = s * RPT
    rem = RPT - (RPT // CH) * CH
    for j in range(RPT // CH):
        pltpu.async_copy(wtd0, acc.at[pl.ds(base + j * CH, CH)], si)
    pltpu.async_copy(wtd0.at[pl.ds(0, rem)],
                     acc.at[pl.ds(base + (RPT // CH) * CH, rem)], si)
    for j in range(RPT // CH):
        pltpu.make_async_copy(wtd0, acc.at[pl.ds(base + j * CH, CH)],
                              si).wait()
    pltpu.make_async_copy(wtd0.at[pl.ds(0, rem)],
                          acc.at[pl.ds(base + (RPT // CH) * CH, rem)],
                          si).wait()
    plsc.subcore_barrier()

    def issue_group(g):
        half = pl.ds((g % 2) * GC, GC)
        pltpu.async_copy(src_hbm.at[wid, g], sidx.at[half], si)
        pltpu.async_copy(dst_hbm.at[wid, g], didx.at[half], si)

    def wait_group(g):
        half = pl.ds((g % 2) * GC, GC)
        pltpu.make_async_copy(src_hbm.at[wid, g], sidx.at[half], si).wait()
        pltpu.make_async_copy(dst_hbm.at[wid, g], didx.at[half], si).wait()

    def issue_gathers(j, p):
        r = irow(j)
        pltpu.async_copy(ht_hbm.at[sidx.at[r]], hta[p], sg[p])
        pltpu.async_copy(asd_hbm.at[didx.at[r]], adr[p], sg[p])

    def wait_gathers(j, p):
        r = irow(j)
        pltpu.make_async_copy(ht_hbm.at[sidx.at[r]], hta[p], sg[p]).wait()
        pltpu.make_async_copy(asd_hbm.at[didx.at[r]], adr[p], sg[p]).wait()

    def issue_scatter(j, p):
        pltpu.async_copy(wtd[p], acc.at[didx.at[irow(j)]], sw[p], add=True)

    def wait_scatter(j, p):
        pltpu.make_async_copy(wtd[p], acc.at[didx.at[irow(j)]], sw[p]).wait()

    def compute(p):
        htap = hta[p]
        adrp = adr[p]
        wtdp = wtd[p]

        @plsc.parallel_loop(0, CH, unroll=5)
        def edge(e):
            av = htap[e, hc:hc + 16]
            dv = _lane_bcast(adrp[e], rot)
            ev = av + dv
            ev = jnp.maximum(ev, NEG * ev)
            exv = jnp.exp(ev)
            wtdp[e, hc:hc + 16] = exv
            for k in range(hc // 16):
                mj = _lane_bcast(exv, _head_idx(k, hc))
                wtdp[e, k * 16:(k + 1) * 16] = \
                    htap[e, k * 16:(k + 1) * 16] * mj

    pltpu.sync_copy(src_hbm.at[wid, 0], sidx.at[pl.ds(0, GC)])
    pltpu.sync_copy(dst_hbm.at[wid, 0], didx.at[pl.ds(0, GC)])
    issue_gathers(0, 0)
    issue_gathers(1, 1)

    def group(g, _):
        for t in range(GC):
            j = g * GC + t
            p = t % 2
            if t == 2:
                @pl.when(g + 1 < NG)
                def _():
                    issue_group(g + 1)

            wait_gathers(j, p)
            if t == GC - 2:
                @pl.when(g + 1 < NG)
                def _():
                    wait_group(g + 1)

            @pl.when(j >= 2)
            def _():
                wait_scatter(j - 2, p)

            compute(p)
            issue_scatter(j, p)

            @pl.when(j + 2 < NCH)
            def _():
                issue_gathers(j + 2, p)
        return 0

    lax.fori_loop(0, NG, group, 0)
    wait_scatter(NCH - 2, 0)
    wait_scatter(NCH - 1, 1)
    plsc.subcore_barrier()
    for j in range(RPT // CH):
        sl = pl.ds(base + j * CH, CH)
        pltpu.async_copy(acc.at[sl], out_hbm.at[c].at[sl], si)
    sl = pl.ds(base + (RPT // CH) * CH, rem)
    pltpu.async_copy(acc.at[sl], out_hbm.at[c].at[sl], si)
    for j in range(RPT // CH):
        sl = pl.ds(base + j * CH, CH)
        pltpu.make_async_copy(acc.at[sl], out_hbm.at[c].at[sl], si).wait()
    sl = pl.ds(base + (RPT // CH) * CH, rem)
    pltpu.make_async_copy(acc.at[sl], out_hbm.at[c].at[sl], si).wait()


def _sc_edge(ht, asd, src, dst):
    hc = ht.shape[1] - 16
    wl = hc + 16
    mesh = plsc.VectorSubcoreMesh(core_axis_name="c", subcore_axis_name="s")
    kern = pl.kernel(
        functools.partial(_sc_body, hc=hc),
        out_type=jax.ShapeDtypeStruct((2, N, wl), jnp.float32),
        mesh=mesh,
        compiler_params=pltpu.CompilerParams(use_tc_tiling_on_sc=False),
        scratch_types=[
            pltpu.VMEM((2 * GC, CH), jnp.int32),
            pltpu.VMEM((2 * GC, CH), jnp.int32),
            pltpu.VMEM((CH, hc + 16), jnp.float32),
            pltpu.VMEM((CH, 16), jnp.float32),
            pltpu.VMEM((CH, wl), jnp.float32),
            pltpu.VMEM((CH, hc + 16), jnp.float32),
            pltpu.VMEM((CH, 16), jnp.float32),
            pltpu.VMEM((CH, wl), jnp.float32),
            pltpu.SemaphoreType.DMA,
            pltpu.SemaphoreType.DMA,
            pltpu.SemaphoreType.DMA,
            pltpu.SemaphoreType.DMA,
            pltpu.SemaphoreType.DMA,
            pltpu.VMEM_SHARED((N, wl), jnp.float32),
        ],
    )
    return kern(ht, asd, src.reshape(NW, NG, GC, CH),
                dst.reshape(NW, NG, GC, CH))


def _block_diag_asd(a_s, a_d):
    hh, cc = a_s.shape
    m = jnp.zeros((hh * cc, 2 * hh), jnp.float32)
    rows = jnp.arange(hh * cc)
    heads = rows // cc
    m = m.at[rows, heads].set(a_s.reshape(-1))
    m = m.at[rows, hh + heads].set(a_d.reshape(-1))
    return m


def _rep_mat(hc):
    return jnp.repeat(jnp.eye(H, dtype=jnp.float32), hc // H, axis=1)


def kernel(x, edge_index, batch, W1, a1s, a1d, b1, W2, a2s, a2d, b2,
           W3, a3s, a3d, b3, bn_g, bn_b, bn_m, bn_v,
           fc1_w, fc1_b, fc2_w, fc2_b):
    src = edge_index[0]
    dst = edge_index[1]
    asd1_w = _block_diag_asd(a1s, a1d)
    asd2_w = _block_diag_asd(a2s, a2d)
    asd3_w = _block_diag_asd(a3s, a3d)
    rm1 = _rep_mat(64)
    rm2 = _rep_mat(128)
    batch2d = batch.reshape(N, 1)

    ht1, asd1 = _tc_front(x, W1, asd1_w)
    p1 = _sc_edge(ht1, asd1, src, dst)
    ht2, asd2 = _tc_mid(p1, ht1, rm1, b1.reshape(1, 64), W2, asd2_w)
    p2 = _sc_edge(ht2, asd2, src, dst)
    ht3, asd3 = _tc_mid(p2, ht2, rm2, b2.reshape(1, 128), W3, asd3_w)
    p3 = _sc_edge(ht3, asd3, src, dst)
    return _tc_pool_head(p3, ht3, rm2, b3.reshape(1, 128), batch2d,
                         bn_m.reshape(1, 128), bn_v.reshape(1, 128),
                         bn_g.reshape(1, 128), bn_b.reshape(1, 128),
                         fc1_w, fc1_b.reshape(1, 32), fc2_w,
                         fc2_b.reshape(1, 10))


# speedup vs baseline: 186.1233x; 1.0897x over previous
import functools
import jax
import jax.numpy as jnp
from jax import lax
from jax.experimental import pallas as pl
from jax.experimental.pallas import tpu as pltpu
from jax.experimental.pallas import tpu_sc as plsc

N = 10000
E = 320000
H = 8
G = 128
NEG = 0.2
NW = 32
CH = 50
NCH = E // (NW * CH)
GC = 10
NG = NCH // GC
RPT = N // 16


def _front_body(x_ref, w_ref, a_ref, ht_ref, asd_ref, *, hc):
    h = jnp.dot(x_ref[...], w_ref[...], preferred_element_type=jnp.float32)
    asd = jnp.dot(h, a_ref[...], preferred_element_type=jnp.float32)
    ht_ref[:, :hc] = h
    ht_ref[:, hc:] = asd
    asd_ref[...] = asd


def _tc_front(x, w, asd_w, bm=2000):
    fin = x.shape[1]
    hc = w.shape[1]
    grid = x.shape[0] // bm
    return pl.pallas_call(
        functools.partial(_front_body, hc=hc),
        grid=(grid,),
        in_specs=[
            pl.BlockSpec((bm, fin), lambda i: (i, 0)),
            pl.BlockSpec((fin, hc), lambda i: (0, 0)),
            pl.BlockSpec((hc, 16), lambda i: (0, 0)),
        ],
        out_specs=[
            pl.BlockSpec((bm, hc + 16), lambda i: (i, 0)),
            pl.BlockSpec((bm, 16), lambda i: (i, 0)),
        ],
        out_shape=[
            jax.ShapeDtypeStruct((x.shape[0], hc + 16), jnp.float32),
            jax.ShapeDtypeStruct((x.shape[0], 16), jnp.float32),
        ],
    )(x, w, asd_w)


def _combine(p0, p1, ht, rm, b, hc):
    h = ht[:, :hc]
    als = ht[:, hc:hc + 8]
    ald = ht[:, hc + 8:hc + 16]
    num = p0[:, :hc] + p1[:, :hc]
    den8 = p0[:, hc:hc + 8] + p1[:, hc:hc + 8]
    es = als + ald
    es = jnp.maximum(es, NEG * es)
    exs = jnp.exp(es)
    exs_e = jnp.dot(exs, rm, preferred_element_type=jnp.float32)
    num = num + exs_e * h
    den = jnp.dot(den8 + exs, rm, preferred_element_type=jnp.float32)
    out = num / (den + 1e-16) + b
    return jnp.where(out > 0, out, jnp.exp(jnp.minimum(out, 0.0)) - 1.0)


def _mid_body(p0_ref, p1_ref, ht_ref, rm_ref, b_ref, w_ref, a_ref,
              htn_ref, asdn_ref, *, hc, hc2):
    act = _combine(p0_ref[...], p1_ref[...], ht_ref[...],
                   rm_ref[...], b_ref[...], hc)
    hn = jnp.dot(act, w_ref[...], preferred_element_type=jnp.float32)
    asdn = jnp.dot(hn, a_ref[...], preferred_element_type=jnp.float32)
    htn_ref[:, :hc2] = hn
    htn_ref[:, hc2:] = asdn
    asdn_ref[...] = asdn


def _tc_mid(p, ht, rm, b, w, asd_w, bm=2000):
    hc = ht.shape[1] - 16
    wl = hc + 16
    hc2 = w.shape[1]
    grid = N // bm
    return pl.pallas_call(
        functools.partial(_mid_body, hc=hc, hc2=hc2),
        grid=(grid,),
        in_specs=[
            pl.BlockSpec((bm, wl), lambda i: (i, 0)),
            pl.BlockSpec((bm, wl), lambda i: (i, 0)),
            pl.BlockSpec((bm, hc + 16), lambda i: (i, 0)),
            pl.BlockSpec((8, hc), lambda i: (0, 0)),
            pl.BlockSpec((1, hc), lambda i: (0, 0)),
            pl.BlockSpec((hc, hc2), lambda i: (0, 0)),
            pl.BlockSpec((hc2, 16), lambda i: (0, 0)),
        ],
        out_specs=[
            pl.BlockSpec((bm, hc2 + 16), lambda i: (i, 0)),
            pl.BlockSpec((bm, 16), lambda i: (i, 0)),
        ],
        out_shape=[
            jax.ShapeDtypeStruct((N, hc2 + 16), jnp.float32),
            jax.ShapeDtypeStruct((N, 16), jnp.float32),
        ],
    )(p[0], p[1], ht, rm, b, w, asd_w)


def _pool_body(p0_ref, p1_ref, ht_ref, rm_ref, b_ref, bt_ref,
               bnm_ref, bnv_ref, bng_ref, bnb_ref,
               f1w_ref, f1b_ref, f2w_ref, f2b_ref,
               out_ref, sum_ref, cnt_ref, *, hc, bm):
    act = _combine(p0_ref[...], p1_ref[...], ht_ref[...],
                   rm_ref[...], b_ref[...], hc)
    gi = lax.broadcasted_iota(jnp.int32, (bm, G), 1)
    oh = (bt_ref[...] == gi).astype(jnp.float32)
    dn = (((0,), (0,)), ((), ()))
    s = lax.dot_general(oh, act, dn, preferred_element_type=jnp.float32)
    c = lax.dot_general(oh, jnp.ones_like(act), dn,
                        preferred_element_type=jnp.float32)
    step = pl.program_id(0)

    @pl.when(step == 0)
    def _():
        sum_ref[...] = s
        cnt_ref[...] = c

    @pl.when(step != 0)
    def _():
        sum_ref[...] += s
        cnt_ref[...] += c

    @pl.when(step == pl.num_programs(0) - 1)
    def _():
        pooled = sum_ref[...] / jnp.maximum(cnt_ref[...], 1.0)
        y = (pooled - bnm_ref[...]) / jnp.sqrt(bnv_ref[...] + 1e-5)
        y = y * bng_ref[...] + bnb_ref[...]
        z = jnp.dot(y, f1w_ref[...], preferred_element_type=jnp.float32)
        z = jnp.maximum(z + f1b_ref[...], 0.0)
        lg = jnp.dot(z, f2w_ref[...], preferred_element_type=jnp.float32)
        lg = lg + f2b_ref[...]
        m = jnp.max(lg, axis=1, keepdims=True)
        lse = jnp.log(jnp.sum(jnp.exp(lg - m), axis=1, keepdims=True))
        out_ref[...] = lg - m - lse


def _tc_pool_head(p, ht, rm, b, batch2d, bn_m, bn_v, bn_g, bn_b,
                  f1w, f1b, f2w, f2b, bm=2000):
    hc = ht.shape[1] - 16
    wl = hc + 16
    grid = N // bm
    cst = lambda i: (0, 0)
    return pl.pallas_call(
        functools.partial(_pool_body, hc=hc, bm=bm),
        grid=(grid,),
        in_specs=[
            pl.BlockSpec((bm, wl), lambda i: (i, 0)),
            pl.BlockSpec((bm, wl), lambda i: (i, 0)),
            pl.BlockSpec((bm, hc + 16), lambda i: (i, 0)),
            pl.BlockSpec((8, hc), cst),
            pl.BlockSpec((1, hc), cst),
            pl.BlockSpec((bm, 1), lambda i: (i, 0)),
            pl.BlockSpec((1, hc), cst),
            pl.BlockSpec((1, hc), cst),
            pl.BlockSpec((1, hc), cst),
            pl.BlockSpec((1, hc), cst),
            pl.BlockSpec((hc, 32), cst),
            pl.BlockSpec((1, 32), cst),
            pl.BlockSpec((32, 10), cst),
            pl.BlockSpec((1, 10), cst),
        ],
        out_specs=pl.BlockSpec((G, 10), cst),
        out_shape=jax.ShapeDtypeStruct((G, 10), jnp.float32),
        scratch_shapes=[
            pltpu.VMEM((G, hc), jnp.float32),
            pltpu.VMEM((G, hc), jnp.float32),
        ],
    )(p[0], p[1], ht, rm, b, batch2d, bn_m, bn_v, bn_g, bn_b,
      f1w, f1b, f2w, f2b)


def _lane_bcast(v, idx):
    dn = lax.GatherDimensionNumbers(
        offset_dims=(), collapsed_slice_dims=(0,), start_index_map=(0,))
    return lax.gather(v, idx[:, None], dn, (1,),
                      mode=lax.GatherScatterMode.PROMISE_IN_BOUNDS)


def _head_idx(j, hc):
    lanes = lax.iota(jnp.int32, 16)
    if hc == 128:
        return jnp.full((16,), j, jnp.int32)
    return 2 * j + (lanes >> 3)


def _sc_body(ht_hbm, asd_hbm, src_hbm, dst_hbm, out_hbm,
             sidx, didx, hta0, adr0, wtd0, hta1, adr1, wtd1,
             sg0, sg1, sw0, sw1, si, acc, *, hc):
    wl = hc + 16
    c = lax.axis_index("c")
    s = lax.axis_index("s")
    wid = s * 2 + c
    lanes = lax.iota(jnp.int32, 16)
    rot = (lanes & 7) + 8
    hta = (hta0, hta1)
    adr = (adr0, adr1)
    wtd = (wtd0, wtd1)
    sg = (sg0, sg1)
    sw = (sw0, sw1)

    def irow(j):
        return ((j // GC) % 2) * GC + (j % GC)

    def zb(r, _):
        for k in range(wl // 16):
            wtd0[r, k * 16:(k + 1) * 16] = jnp.zeros((16,), jnp.float32)
        return 0

    lax.fori_loop(0, CH, zb, 0)
    base = s * RPT
    rem = RPT - (RPT // CH) * CH
    for j in range(RPT // CH):
        pltpu.async_copy(wtd0, acc.at[pl.ds(base + j * CH, CH)], si)
    pltpu.async_copy(wtd0.at[pl.ds(0, rem)],
                     acc.at[pl.ds(base + (RPT // CH) * CH, rem)], si)
    for j in range(RPT // CH):
        pltpu.make_async_copy(wtd0, acc.at[pl.ds(base + j * CH, CH)],
                              si).wait()
    pltpu.make_async_copy(wtd0.at[pl.ds(0, rem)],
                          acc.at[pl.ds(base + (RPT // CH) * CH, rem)],
                          si).wait()
    plsc.subcore_barrier()

    def issue_group(g):
        half = pl.ds((g % 2) * GC, GC)
        pltpu.async_copy(src_hbm.at[wid, g], sidx.at[half], si)
        pltpu.async_copy(dst_hbm.at[wid, g], didx.at[half], si)

    def wait_group(g):
        half = pl.ds((g % 2) * GC, GC)
        pltpu.make_async_copy(src_hbm.at[wid, g], sidx.at[half], si).wait()
        pltpu.make_async_copy(dst_hbm.at[wid, g], didx.at[half], si).wait()

    def issue_gathers(j, p):
        r = irow(j)
        pltpu.async_copy(ht_hbm.at[sidx.at[r]], hta[p], sg[p])
        pltpu.async_copy(asd_hbm.at[didx.at[r]], adr[p], sg[p])

    def wait_gathers(j, p):
        r = irow(j)
        pltpu.make_async_copy(ht_hbm.at[sidx.at[r]], hta[p], sg[p]).wait()
        pltpu.make_async_copy(asd_hbm.at[didx.at[r]], adr[p], sg[p]).wait()

    def issue_scatter(j, p):
        pltpu.async_copy(wtd[p], acc.at[didx.at[irow(j)]], sw[p], add=True)

    def wait_scatter(j, p):
        pltpu.make_async_copy(wtd[p], acc.at[didx.at[irow(j)]], sw[p]).wait()

    def compute(p):
        htap = hta[p]
        adrp = adr[p]
        wtdp = wtd[p]

        @plsc.parallel_loop(0, CH, unroll=1)
        def edge(e):
            av = htap[e, hc:hc + 16]
            dv = _lane_bcast(adrp[e], rot)
            ev = av + dv
            ev = jnp.maximum(ev, NEG * ev)
            exv = jnp.exp(ev)
            wtdp[e, hc:hc + 16] = exv
            for k in range(hc // 16):
                mj = _lane_bcast(exv, _head_idx(k, hc))
                wtdp[e, k * 16:(k + 1) * 16] = \
                    htap[e, k * 16:(k + 1) * 16] * mj

    pltpu.sync_copy(src_hbm.at[wid, 0], sidx.at[pl.ds(0, GC)])
    pltpu.sync_copy(dst_hbm.at[wid, 0], didx.at[pl.ds(0, GC)])
    issue_gathers(0, 0)
    issue_gathers(1, 1)

    def group(g, _):
        for t in range(GC):
            j = g * GC + t
            p = t % 2
            if t == 2:
                @pl.when(g + 1 < NG)
                def _():
                    issue_group(g + 1)

            wait_gathers(j, p)
            if t == GC - 2:
                @pl.when(g + 1 < NG)
                def _():
                    wait_group(g + 1)

            @pl.when(j >= 2)
            def _():
                wait_scatter(j - 2, p)

            compute(p)
            issue_scatter(j, p)

            @pl.when(j + 2 < NCH)
            def _():
                issue_gathers(j + 2, p)
        return 0

    lax.fori_loop(0, NG, group, 0)
    wait_scatter(NCH - 2, 0)
    wait_scatter(NCH - 1, 1)
    plsc.subcore_barrier()
    for j in range(RPT // CH):
        sl = pl.ds(base + j * CH, CH)
        pltpu.async_copy(acc.at[sl], out_hbm.at[c].at[sl], si)
    sl = pl.ds(base + (RPT // CH) * CH, rem)
    pltpu.async_copy(acc.at[sl], out_hbm.at[c].at[sl], si)
    for j in range(RPT // CH):
        sl = pl.ds(base + j * CH, CH)
        pltpu.make_async_copy(acc.at[sl], out_hbm.at[c].at[sl], si).wait()
    sl = pl.ds(base + (RPT // CH) * CH, rem)
    pltpu.make_async_copy(acc.at[sl], out_hbm.at[c].at[sl], si).wait()


def _sc_edge(ht, asd, src, dst):
    hc = ht.shape[1] - 16
    wl = hc + 16
    mesh = plsc.VectorSubcoreMesh(core_axis_name="c", subcore_axis_name="s")
    kern = pl.kernel(
        functools.partial(_sc_body, hc=hc),
        out_type=jax.ShapeDtypeStruct((2, N, wl), jnp.float32),
        mesh=mesh,
        compiler_params=pltpu.CompilerParams(use_tc_tiling_on_sc=False),
        scratch_types=[
            pltpu.VMEM((2 * GC, CH), jnp.int32),
            pltpu.VMEM((2 * GC, CH), jnp.int32),
            pltpu.VMEM((CH, hc + 16), jnp.float32),
            pltpu.VMEM((CH, 16), jnp.float32),
            pltpu.VMEM((CH, wl), jnp.float32),
            pltpu.VMEM((CH, hc + 16), jnp.float32),
            pltpu.VMEM((CH, 16), jnp.float32),
            pltpu.VMEM((CH, wl), jnp.float32),
            pltpu.SemaphoreType.DMA,
            pltpu.SemaphoreType.DMA,
            pltpu.SemaphoreType.DMA,
            pltpu.SemaphoreType.DMA,
            pltpu.SemaphoreType.DMA,
            pltpu.VMEM_SHARED((N, wl), jnp.float32),
        ],
    )
    return kern(ht, asd, src.reshape(NW, NG, GC, CH),
                dst.reshape(NW, NG, GC, CH))


def _block_diag_asd(a_s, a_d):
    hh, cc = a_s.shape
    m = jnp.zeros((hh * cc, 2 * hh), jnp.float32)
    rows = jnp.arange(hh * cc)
    heads = rows // cc
    m = m.at[rows, heads].set(a_s.reshape(-1))
    m = m.at[rows, hh + heads].set(a_d.reshape(-1))
    return m


def _rep_mat(hc):
    return jnp.repeat(jnp.eye(H, dtype=jnp.float32), hc // H, axis=1)


def kernel(x, edge_index, batch, W1, a1s, a1d, b1, W2, a2s, a2d, b2,
           W3, a3s, a3d, b3, bn_g, bn_b, bn_m, bn_v,
           fc1_w, fc1_b, fc2_w, fc2_b):
    src = edge_index[0]
    dst = edge_index[1]
    asd1_w = _block_diag_asd(a1s, a1d)
    asd2_w = _block_diag_asd(a2s, a2d)
    asd3_w = _block_diag_asd(a3s, a3d)
    rm1 = _rep_mat(64)
    rm2 = _rep_mat(128)
    batch2d = batch.reshape(N, 1)

    ht1, asd1 = _tc_front(x, W1, asd1_w)
    p1 = _sc_edge(ht1, asd1, src, dst)
    ht2, asd2 = _tc_mid(p1, ht1, rm1, b1.reshape(1, 64), W2, asd2_w)
    p2 = _sc_edge(ht2, asd2, src, dst)
    ht3, asd3 = _tc_mid(p2, ht2, rm2, b2.reshape(1, 128), W3, asd3_w)
    p3 = _sc_edge(ht3, asd3, src, dst)
    return _tc_pool_head(p3, ht3, rm2, b3.reshape(1, 128), batch2d,
                         bn_m.reshape(1, 128), bn_v.reshape(1, 128),
                         bn_g.reshape(1, 128), bn_b.reshape(1, 128),
                         fc1_w, fc1_b.reshape(1, 32), fc2_w,
                         fc2_b.reshape(1, 10))
